# Initial kernel scaffold; baseline (speedup 1.0000x reference)
import jax
import jax.numpy as jnp
from jax.experimental import pallas as pl


def kernel(x, edge_index, batch, W1, b1, g1, be1, W2, b2, g2, be2, Wc, bc):
    raise NotImplementedError("write your pallas kernel here")


# speedup vs baseline: 10.1367x; 10.1367x over previous
import functools

import jax
import jax.numpy as jnp
from jax import lax
from jax.experimental import pallas as pl
from jax.experimental.pallas import tpu as pltpu
from jax.experimental.pallas import tpu_sc as plsc

N = 10000
E = 320000
D_IN = 128
D_H = 256
DH2 = 128
N_CLASSES = 10
N_GRAPHS = 64
EPS = 1e-5

NP = 10112
NT = 16
NC = 2
RPT = NP // NT
CH = 128
K = 160
E_PAD = NT * K * CH
K_DEG = E_PAD // (NC * NT * CH)


def _mesh():
    return plsc.VectorSubcoreMesh(core_axis_name="c", subcore_axis_name="s")


def _deg_body(dsts_hbm, ones_hbm, zeros_hbm, deg_hbm,
              dst_v, ones_v, dacc, sem):
    c = lax.axis_index("c")
    s = lax.axis_index("s")
    w = c * NT + s
    pltpu.sync_copy(dsts_hbm.at[pl.ds(w * K_DEG, K_DEG)], dst_v)
    pltpu.sync_copy(ones_hbm, ones_v)
    r0 = s * RPT
    pltpu.sync_copy(zeros_hbm.at[pl.ds(r0, RPT)], dacc.at[pl.ds(r0, RPT)])
    plsc.subcore_barrier()

    def step(j, carry):
        pltpu.sync_copy(ones_v, dacc.at[dst_v.at[j]], add=True)
        return carry

    lax.fori_loop(0, K_DEG, step, 0)
    plsc.subcore_barrier()
    pltpu.sync_copy(dacc.at[pl.ds(r0, RPT)], deg_hbm.at[pl.ds(c * NP + r0, RPT)])


@functools.partial(jax.jit, static_argnums=())
def _deg_call(dsts, ones_d, zeros_d):
    f = pl.kernel(
        _deg_body,
        out_type=jax.ShapeDtypeStruct((NC * NP, 16), jnp.float32),
        mesh=_mesh(),
        scratch_types=[
            pltpu.VMEM((K_DEG, CH), jnp.int32),
            pltpu.VMEM((CH, 16), jnp.float32),
            pltpu.VMEM_SHARED((NP, 16), jnp.float32),
            pltpu.SemaphoreType.DMA,
        ],
    )
    return f(dsts, ones_d, zeros_d)


G = 8
NGRP = K // G
NPAIR = NGRP // 2


def _agg_body(hs_hbm, srcs_hbm, dsts_hbm, out_hbm,
              src_a, dst_a, src_b, dst_b, buf0, buf1, acc,
              sem_a, sem_b, sem0, sem1):
    c = lax.axis_index("c")
    s = lax.axis_index("s")
    w = c * NT + s
    base_s = w * K
    base_d = s * K
    r0 = s * RPT
    pltpu.sync_copy(hs_hbm.at[pl.ds(c * NP + r0, RPT)], acc.at[pl.ds(r0, RPT)])
    pltpu.async_copy(srcs_hbm.at[pl.ds(base_s, G)], src_a, sem_a)
    pltpu.async_copy(dsts_hbm.at[pl.ds(base_d, G)], dst_a, sem_a)
    pltpu.async_copy(srcs_hbm.at[pl.ds(base_s + G, G)], src_b, sem_b)
    pltpu.async_copy(dsts_hbm.at[pl.ds(base_d + G, G)], dst_b, sem_b)
    plsc.subcore_barrier()

    def wait_idx(src_v, dst_v, sem):
        pltpu.make_async_copy(srcs_hbm.at[pl.ds(base_s, G)], src_v, sem).wait()
        pltpu.make_async_copy(dsts_hbm.at[pl.ds(base_d, G)], dst_v, sem).wait()

    def gather(idx_row, buf, sem):
        pltpu.async_copy(hs_hbm.at[idx_row], buf, sem)

    def wait_gather(buf, sem):
        pltpu.make_async_copy(hs_hbm.at[src_a.at[0]], buf, sem).wait()

    wait_idx(src_a, dst_a, sem_a)
    gather(src_a.at[0], buf0, sem0)

    def pair(p, carry):
        for k in range(2 * G):
            srow = (src_a if k < G else src_b).at[k % G]
            drow = (dst_a if k < G else dst_b).at[k % G]
            buf, sem = (buf0, sem0) if k % 2 == 0 else (buf1, sem1)
            nbuf, nsem = (buf1, sem1) if k % 2 == 0 else (buf0, sem0)
            if k + 1 == G:
                wait_idx(src_b, dst_b, sem_b)
            if k + 1 < 2 * G:
                nrow = (src_a if k + 1 < G else src_b).at[(k + 1) % G]
                gather(nrow, nbuf, nsem)
            else:
                @pl.when(p < NPAIR - 1)
                def _():
                    wait_idx(src_a, dst_a, sem_a)
                    gather(src_a.at[0], nbuf, nsem)
            wait_gather(buf, sem)
            pltpu.sync_copy(buf, acc.at[drow], add=True)
            if k == G - 1:
                @pl.when(p < NPAIR - 1)
                def _():
                    g2 = (2 * p + 2) * G
                    pltpu.async_copy(srcs_hbm.at[pl.ds(base_s + g2, G)], src_a, sem_a)
                    pltpu.async_copy(dsts_hbm.at[pl.ds(base_d + g2, G)], dst_a, sem_a)
            if k == 2 * G - 1:
                @pl.when(p < NPAIR - 1)
                def _():
                    g3 = (2 * p + 3) * G
                    pltpu.async_copy(srcs_hbm.at[pl.ds(base_s + g3, G)], src_b, sem_b)
                    pltpu.async_copy(dsts_hbm.at[pl.ds(base_d + g3, G)], dst_b, sem_b)
        return carry

    lax.fori_loop(0, NPAIR, pair, 0)
    plsc.subcore_barrier()
    pltpu.sync_copy(acc.at[pl.ds(r0, RPT)], out_hbm.at[pl.ds(c * NP + r0, RPT)])


def _agg_call(hs_flat, srcs, dsts):
    f = pl.kernel(
        _agg_body,
        out_type=jax.ShapeDtypeStruct((NC * NP, DH2), jnp.float32),
        mesh=_mesh(),
        scratch_types=[
            pltpu.VMEM((G, CH), jnp.int32),
            pltpu.VMEM((G, CH), jnp.int32),
            pltpu.VMEM((G, CH), jnp.int32),
            pltpu.VMEM((G, CH), jnp.int32),
            pltpu.VMEM((CH, DH2), jnp.float32),
            pltpu.VMEM((CH, DH2), jnp.float32),
            pltpu.VMEM_SHARED((NP, DH2), jnp.float32),
            pltpu.SemaphoreType.DMA,
            pltpu.SemaphoreType.DMA,
            pltpu.SemaphoreType.DMA,
            pltpu.SemaphoreType.DMA,
        ],
    )
    return f(hs_flat, srcs, dsts)


def _dinv_from_parts(degp_ref):
    deg = degp_ref[0] + degp_ref[1] + 1.0
    rows = lax.broadcasted_iota(jnp.int32, (NP, 1), 0)
    return jnp.where(rows < N, lax.rsqrt(deg[:, :1]), 0.0)


def _tc1_body(x_ref, w1_ref, degp_ref, hs_ref):
    dinv = _dinv_from_parts(degp_ref)
    h = jnp.dot(x_ref[...], w1_ref[...], preferred_element_type=jnp.float32)
    hs = h * dinv
    hs_ref[0] = hs[:, :DH2]
    hs_ref[1] = hs[:, DH2:]


def _bn_relu(z, g, be):
    m = jnp.sum(z, axis=0, keepdims=True) * (1.0 / N)
    v = jnp.sum(z * z, axis=0, keepdims=True) * (1.0 / N) - m * m
    return jnp.maximum((z - m) * lax.rsqrt(v + EPS) * g + be, 0.0)


def _tc2_body(acc_ref, degp_ref, g_ref, be_ref, w2_ref, hs2_ref):
    dinv = _dinv_from_parts(degp_ref)
    r0 = _bn_relu(acc_ref[0] * dinv, g_ref[:, :DH2], be_ref[:, :DH2])
    r1 = _bn_relu(acc_ref[1] * dinv, g_ref[:, DH2:], be_ref[:, DH2:])
    h2 = (jnp.dot(r0, w2_ref[:DH2, :], preferred_element_type=jnp.float32)
          + jnp.dot(r1, w2_ref[DH2:, :], preferred_element_type=jnp.float32))
    hs2 = h2 * dinv
    hs2_ref[0] = hs2[:, :DH2]
    hs2_ref[1] = hs2[:, DH2:]


def _tc3_body(acc_ref, degp_ref, g_ref, be_ref, batch_ref, wc_ref, bc_ref,
              out_ref):
    dinv = _dinv_from_parts(degp_ref)
    r0 = _bn_relu(acc_ref[0] * dinv, g_ref[:, :DH2], be_ref[:, :DH2])
    r1 = _bn_relu(acc_ref[1] * dinv, g_ref[:, DH2:], be_ref[:, DH2:])
    gids = lax.broadcasted_iota(jnp.int32, (N_GRAPHS, NP), 0)
    m = jnp.where(batch_ref[...] == gids, 1.0, 0.0)
    p0 = jnp.dot(m, r0, preferred_element_type=jnp.float32)
    p1 = jnp.dot(m, r1, preferred_element_type=jnp.float32)
    out_ref[...] = (jnp.dot(p0, wc_ref[:DH2, :], preferred_element_type=jnp.float32)
                    + jnp.dot(p1, wc_ref[DH2:, :], preferred_element_type=jnp.float32)
                    + bc_ref[...])


def _tc1_call(x_pad, w1, degp):
    return pl.pallas_call(
        _tc1_body,
        out_shape=jax.ShapeDtypeStruct((NC, NP, DH2), jnp.float32),
    )(x_pad, w1, degp)


def _tc2_call(acc, degp, g, be, w2):
    return pl.pallas_call(
        _tc2_body,
        out_shape=jax.ShapeDtypeStruct((NC, NP, DH2), jnp.float32),
    )(acc, degp, g, be, w2)


def _tc3_call(acc, degp, g, be, batch2d, wc, bc):
    return pl.pallas_call(
        _tc3_body,
        out_shape=jax.ShapeDtypeStruct((N_GRAPHS, N_CLASSES), jnp.float32),
    )(acc, degp, g, be, batch2d, wc, bc)


def kernel(x, edge_index, batch, W1, b1, g1, be1, W2, b2, g2, be2, Wc, bc):
    src = edge_index[0]
    dst = edge_index[1]
    pad = E_PAD - E
    srcp = jnp.concatenate([src, jnp.full((pad,), N, jnp.int32)])
    dstp = jnp.concatenate([dst, jnp.full((pad,), N, jnp.int32)])
    srcs = jnp.stack([srcp, srcp + NP]).reshape(NC * NT * K, CH)
    dsts = dstp.reshape(NT * K, CH)
    dstd = dstp.reshape(NC * NT * K_DEG, CH)
    x_pad = jnp.pad(x, ((0, NP - N), (0, 0)))
    batch2d = jnp.pad(batch, (0, NP - N), constant_values=N_GRAPHS).reshape(1, NP)
    ones_d = jnp.ones((CH, 16), jnp.float32)
    zeros_d = jnp.zeros((NP, 16), jnp.float32)

    degp = _deg_call(dstd, ones_d, zeros_d).reshape(NC, NP, 16)
    hs1 = _tc1_call(x_pad, W1, degp)
    acc1 = _agg_call(hs1.reshape(NC * NP, DH2), srcs, dsts).reshape(NC, NP, DH2)
    hs2 = _tc2_call(acc1, degp, g1.reshape(1, D_H), be1.reshape(1, D_H), W2)
    acc2 = _agg_call(hs2.reshape(NC * NP, DH2), srcs, dsts).reshape(NC, NP, DH2)
    return _tc3_call(acc2, degp, g2.reshape(1, D_H), be2.reshape(1, D_H),
                     batch2d, Wc, bc.reshape(1, N_CLASSES))
